# baseline (speedup 1.0000x reference)
import functools

import jax
import jax.numpy as jnp
from jax import lax
from jax.experimental import pallas as pl
from jax.experimental.pallas import tpu as pltpu
from jax.experimental.pallas import tpu_sc as plsc

NROW = 5
DIM = 256
BN = 16384

_info = plsc.get_sparse_core_info()
_NC, _NS = _info.num_cores, _info.num_subcores
_NW = _NC * _NS
_B_PER_W = BN // _NW
_CHUNK = 64
_NCHUNK = _B_PER_W // _CHUNK
_NBUF = 6
_NCOL = DIM // 16

_mesh = plsc.VectorSubcoreMesh(core_axis_name="c", subcore_axis_name="s")


@functools.partial(
    pl.kernel,
    mesh=_mesh,
    out_type=jax.ShapeDtypeStruct((BN, DIM), jnp.float32),
    compiler_params=pltpu.CompilerParams(needs_layout_passes=False),
    scratch_types=[
        pltpu.VMEM((NROW, DIM), jnp.float32),
        pltpu.VMEM((_NCHUNK, _CHUNK), jnp.int32),
        pltpu.VMEM((_NBUF, _CHUNK, DIM), jnp.float32),
        pltpu.SemaphoreType.DMA,
        pltpu.SemaphoreType.DMA,
        pltpu.SemaphoreType.DMA,
        pltpu.SemaphoreType.DMA,
        pltpu.SemaphoreType.DMA,
        pltpu.SemaphoreType.DMA,
        pltpu.SemaphoreType.DMA,
    ],
)
def _embed_build(idx_hbm, table_hbm, out_hbm, table_v, idx_v, rows_v,
                 s0, s1, s2, s3, s4, s5, stg):
    ssems = (s0, s1, s2, s3, s4, s5)
    wid = lax.axis_index("s") * _NC + lax.axis_index("c")
    base = wid * _B_PER_W
    tcp = pltpu.async_copy(table_hbm, table_v, stg)
    icp = pltpu.async_copy(idx_hbm.at[wid], idx_v, stg)
    tcp.wait()
    icp.wait()

    cols = [lax.iota(jnp.int32, 16) + 16 * j for j in range(_NCOL)]

    def build_chunk(ch, buf):
        chv = jnp.full((16,), ch, jnp.int32)

        @plsc.parallel_loop(0, _CHUNK, step=1, unroll=8)
        def row_body(r):
            rv = jnp.full((16,), r, jnp.int32)
            row_splat = plsc.load_gather(idx_v, [chv, rv])
            for j in range(_NCOL):
                val = plsc.load_gather(table_v, [row_splat, cols[j]])
                plsc.store_scatter(rows_v.at[buf], [rv, cols[j]], val)

    def scatter(ch):
        return pltpu.async_copy(
            rows_v.at[ch % _NBUF],
            out_hbm.at[pl.ds(base + ch * _CHUNK, _CHUNK)],
            ssems[ch % _NBUF])

    ss = {}
    for ch in range(_NCHUNK):
        buf = ch % _NBUF
        if ch >= _NBUF:
            ss.pop(ch - _NBUF).wait()
        build_chunk(ch, buf)
        ss[ch] = scatter(ch)
    for ch in sorted(ss):
        ss[ch].wait()


def kernel(cam_indices, source_embed):
    idx = cam_indices.astype(jnp.int32).reshape(_NW, _NCHUNK, _CHUNK)
    return _embed_build(idx, source_embed)


# speedup vs baseline: 1.0874x; 1.0874x over previous
import functools

import jax
import jax.numpy as jnp
from jax import lax
from jax.experimental import pallas as pl
from jax.experimental.pallas import tpu as pltpu
from jax.experimental.pallas import tpu_sc as plsc

NROW = 5
DIM = 256
BN = 16384

_info = plsc.get_sparse_core_info()
_NC, _NS = _info.num_cores, _info.num_subcores
_NW = _NC * _NS
_B_PER_W = BN // _NW
_CHUNK = 128
_NCHUNK = _B_PER_W // _CHUNK
_NBUF = 3
_NCOL = DIM // 16

_mesh = plsc.VectorSubcoreMesh(core_axis_name="c", subcore_axis_name="s")


@functools.partial(
    pl.kernel,
    mesh=_mesh,
    out_type=jax.ShapeDtypeStruct((BN, DIM), jnp.float32),
    compiler_params=pltpu.CompilerParams(needs_layout_passes=False),
    scratch_types=[
        pltpu.VMEM((NROW, DIM), jnp.float32),
        pltpu.VMEM((_NCHUNK, _CHUNK), jnp.int32),
        pltpu.VMEM((_NBUF, _CHUNK, DIM), jnp.float32),
        pltpu.SemaphoreType.DMA,
        pltpu.SemaphoreType.DMA,
        pltpu.SemaphoreType.DMA,
        pltpu.SemaphoreType.DMA,
    ],
)
def _embed_build(idx_hbm, table_hbm, out_hbm, table_v, idx_v, rows_v,
                 s0, s1, s2, stg):
    ssems = (s0, s1, s2)
    wid = lax.axis_index("s") * _NC + lax.axis_index("c")
    base = wid * _B_PER_W
    tcp = pltpu.async_copy(table_hbm, table_v, stg)
    icp = pltpu.async_copy(idx_hbm.at[wid], idx_v, stg)
    tcp.wait()
    icp.wait()

    cols = [lax.iota(jnp.int32, 16) + 16 * j for j in range(_NCOL)]

    def build_chunk(ch, buf):
        chv = jnp.full((16,), ch, jnp.int32)

        @plsc.parallel_loop(0, _CHUNK, step=1, unroll=4)
        def row_body(r):
            rv = jnp.full((16,), r, jnp.int32)
            row_splat = plsc.load_gather(idx_v, [chv, rv])
            for j in range(_NCOL):
                val = plsc.load_gather(table_v, [row_splat, cols[j]])
                plsc.store_scatter(rows_v.at[buf], [rv, cols[j]], val)

    def scatter(ch):
        return pltpu.async_copy(
            rows_v.at[ch % _NBUF],
            out_hbm.at[pl.ds(base + ch * _CHUNK, _CHUNK)],
            ssems[ch % _NBUF])

    ss = {}
    for ch in range(_NCHUNK):
        buf = ch % _NBUF
        if ch >= _NBUF:
            ss.pop(ch - _NBUF).wait()
        build_chunk(ch, buf)
        ss[ch] = scatter(ch)
    for ch in sorted(ss):
        ss[ch].wait()


def kernel(cam_indices, source_embed):
    idx = cam_indices.astype(jnp.int32).reshape(_NW, _NCHUNK, _CHUNK)
    return _embed_build(idx, source_embed)
